# baseline (speedup 1.0000x reference)
import jax
import jax.numpy as jnp
import numpy as np
from jax import lax
from jax.experimental import pallas as pl
from jax.experimental.pallas import tpu as pltpu
from jax.experimental.pallas import tpu_sc as plsc

_MARGIN = 0.1
_N = 16384
_PAIRS = 8192
_LANES = 16
_NT = 16
_CHUNK = _N // _NT
_SPT = _PAIRS // _NT


def _rotl32(x, r):
    return ((x << np.uint32(r)) | (x >> np.uint32(32 - r))).astype(np.uint32)


def _threefry2x32(k1, k2, x0, x1):
    x0 = x0.astype(np.uint32).copy()
    x1 = x1.astype(np.uint32).copy()
    ks = [np.uint32(k1), np.uint32(k2),
          np.uint32(np.uint32(k1) ^ np.uint32(k2) ^ np.uint32(0x1BD11BDA))]
    rotations = [[13, 15, 26, 6], [17, 29, 16, 24]]
    x0 += ks[0]
    x1 += ks[1]
    for i in range(5):
        for r in rotations[i % 2]:
            x0 += x1
            x1 = _rotl32(x1, r)
            x1 ^= x0
        x0 += ks[(i + 1) % 3]
        x1 += np.uint32(ks[(i + 2) % 3] + np.uint32(i + 1))
    return x0, x1


def _fry_bits(k, n):
    i = np.arange(n, dtype=np.uint64)
    o0, o1 = _threefry2x32(k[0], k[1], (i >> np.uint64(32)).astype(np.uint32),
                           (i & np.uint64(0xFFFFFFFF)).astype(np.uint32))
    return o0 ^ o1


def _fry_split(k):
    o0, o1 = _threefry2x32(k[0], k[1], np.zeros(2, np.uint32),
                           np.arange(2, dtype=np.uint32))
    return (o0[0], o1[0]), (o0[1], o1[1])


def _sample_bits():
    sk1, sk2 = _fry_split((np.uint32(0), np.uint32(42)))
    out = []
    for k in (sk1, sk2):
        ra, rb = _fry_split(k)
        for kk in (ra, rb):
            out.append(_fry_bits(kk, _PAIRS).view(np.int32))
    return tuple(out)


def _packed_bits():
    ue, ve, un, vn = _sample_bits()
    tiles = []
    for w in range(_NT):
        sl = slice(w * _SPT, (w + 1) * _SPT)
        tiles.append(np.concatenate([ue[sl], ve[sl], un[sl], vn[sl]]))
    return np.concatenate(tiles)


_RAW_BITS = _packed_bits()


def _vmod(x, span_v, rinv_v):
    q1 = (x.astype(jnp.float32) * rinv_v).astype(jnp.int32)
    r = x - q1 * span_v
    q2 = (r.astype(jnp.float32) * rinv_v).astype(jnp.int32)
    r = r - q2 * span_v
    r = jnp.where(r >= span_v, r - span_v, r)
    r = jnp.where(r < 0, r + span_v, r)
    r = jnp.where(r < 0, r + span_v, r)
    return r


def _ridx(u, v, bf, bh, bg, span_v, rinv_v):
    uhi = lax.shift_right_logical(u, 16)
    ulo = lax.bitwise_and(u, 0xFFFF)
    vhi = lax.shift_right_logical(v, 16)
    vlo = lax.bitwise_and(v, 0xFFFF)
    p1 = _vmod(uhi * bg + ulo * bh, span_v, rinv_v)
    r = _vmod(p1 + vhi * bf + vlo, span_v, rinv_v)
    return jnp.minimum(jnp.maximum(r, 0), span_v - 1)


def _chunk_of(a, ce_tab):
    w = a * 0
    for s in (8, 4, 2, 1):
        probe = plsc.load_gather(ce_tab, [w + (s - 1)])
        w = w + jnp.where(probe <= a, s, 0)
    return w


def _body(probs_hbm, tgt_hbm, bits_hbm, out_hbm,
          pv, tv, locb, cntw, bits, cnts, ce_tab, cx_tab, cen_tab, cxn_tab,
          pos_loc, buf, posf, outv, sem_bits, sem_buf, sem_in,
          sh_data, sh_cnt, sh_pos):
    cid = lax.axis_index("c")
    sid = lax.axis_index("s")

    @pl.when(cid == 0)
    def _():
        lane = lax.iota(jnp.int32, _LANES)

        cp_bits = pltpu.async_copy(bits_hbm.at[pl.ds(sid * 4 * _SPT, 4 * _SPT)],
                                   bits, sem_bits)

        base = sid * _CHUNK
        cp_p = pltpu.async_copy(probs_hbm.at[pl.ds(base, _CHUNK)], pv, sem_in)
        cp_t = pltpu.async_copy(tgt_hbm.at[pl.ds(base, _CHUNK)], tv, sem_in)
        cp_p.wait()
        cp_t.wait()

        def compact4(i, off_v):
            for k in range(8):
                j = i * 8 + k
                sl = pl.ds(j * _LANES, _LANES)
                t = tv[sl]
                p = pv[sl]
                m32 = jnp.where(t == 1, 1, 0)
                rank_e = plsc.cumsum(m32) - m32
                pc = plsc.all_reduce_population_count(t == 1)
                dest = jnp.where(m32 == 1, off_v + rank_e,
                                 (_CHUNK + j * _LANES) + lane - off_v - rank_e)
                plsc.store_scatter(locb, [dest], p)
                off_v = off_v + pc
            return off_v

        off_v = lax.fori_loop(0, _CHUNK // _LANES // 8, compact4,
                              jnp.zeros((_LANES,), jnp.int32))
        pltpu.sync_copy(locb, sh_data.at[pl.ds(sid * 2 * _CHUNK, 2 * _CHUNK)])
        cntw[...] = off_v
        pltpu.sync_copy(cntw, sh_cnt.at[sid])
        plsc.subcore_barrier()

        @pl.when(sid == 0)
        def _():
            pltpu.async_copy(sh_data, buf, sem_buf)

        pltpu.sync_copy(sh_cnt, cnts)
        ce_vec = plsc.load_gather(cnts, [lane, lane * 0])
        ce_inc = plsc.cumsum(ce_vec)
        cx_exc = ce_inc - ce_vec
        cn_vec = _CHUNK - ce_vec
        cn_inc = plsc.cumsum(cn_vec)
        cxn_exc = cn_inc - cn_vec
        ce_tab[...] = ce_inc
        cx_tab[...] = cx_exc
        cen_tab[...] = cn_inc
        cxn_tab[...] = cxn_exc
        n_ess = ce_inc[_LANES - 1]
        n_non = _N - n_ess

        cp_bits.wait()

        se_v = lane * 0 + jnp.maximum(n_ess, 1)
        sn_v = lane * 0 + jnp.maximum(n_non, 1)
        rinv_e = 1.0 / se_v.astype(jnp.float32)
        rinv_n = 1.0 / sn_v.astype(jnp.float32)
        c64k = jnp.full((_LANES,), 65536, jnp.int32)
        bf_e = _vmod(c64k, se_v, rinv_e)
        bh_e = _vmod(bf_e * bf_e, se_v, rinv_e)
        bg_e = _vmod(lax.shift_left(bh_e, 16), se_v, rinv_e)
        bf_n = _vmod(c64k, sn_v, rinv_n)
        bh_n = _vmod(bf_n * bf_n, sn_v, rinv_n)
        bg_n = _vmod(lax.shift_left(bh_n, 16), sn_v, rinv_n)

        def samp4(i, carry):
            for k in range(4):
                v = i * 4 + k
                sl0 = pl.ds(v * _LANES, _LANES)
                a = _ridx(bits[sl0], bits[pl.ds(_SPT + v * _LANES, _LANES)],
                          bf_e, bh_e, bg_e, se_v, rinv_e)
                b = _ridx(bits[pl.ds(2 * _SPT + v * _LANES, _LANES)],
                          bits[pl.ds(3 * _SPT + v * _LANES, _LANES)],
                          bf_n, bh_n, bg_n, sn_v, rinv_n)
                wa = _chunk_of(a, ce_tab)
                pos_a = lax.shift_left(wa, 11) + a - plsc.load_gather(cx_tab, [wa])
                wb = _chunk_of(b, cen_tab)
                pos_b = (lax.shift_left(wb, 11) + _CHUNK + b
                         - plsc.load_gather(cxn_tab, [wb]))
                pos_loc[sl0] = pos_a
                pos_loc[pl.ds(_SPT + v * _LANES, _LANES)] = pos_b
            return carry

        lax.fori_loop(0, _SPT // _LANES // 4, samp4, jnp.int32(0))
        pltpu.sync_copy(pos_loc, sh_pos.at[pl.ds(sid * 2 * _SPT, 2 * _SPT)])
        plsc.subcore_barrier()

        @pl.when(sid == 0)
        def _():
            rest = pl.ds(2 * _SPT, (_NT - 1) * 2 * _SPT)
            cp_pos = pltpu.async_copy(sh_pos.at[rest], posf.at[rest], sem_in)
            pltpu.make_async_copy(sh_data, buf, sem_buf).wait()

            def mk_hinge8(src, rbase):
                def hinge8(i, a_):
                    for k in range(8):
                        o = (i * 8 + k) * _LANES
                        ra = src[pl.ds(rbase + o, _LANES)]
                        rb = src[pl.ds(rbase + _SPT + o, _LANES)]
                        pe = plsc.load_gather(buf, [ra])
                        pn = plsc.load_gather(buf, [rb])
                        a_ = a_ + jnp.maximum(pn - pe + _MARGIN, 0.0)
                    return a_
                return hinge8

            acc = lax.fori_loop(0, _SPT // _LANES // 8,
                                mk_hinge8(pos_loc, 0),
                                jnp.zeros((_LANES,), jnp.float32))
            cp_pos.wait()
            for w in range(1, _NT):
                acc = lax.fori_loop(0, _SPT // _LANES // 8,
                                    mk_hinge8(posf, w * 2 * _SPT), acc)
            mean = jnp.sum(acc) * (1.0 / _PAIRS)
            ok = jnp.logical_and(n_ess > 0, n_non > 0)
            res = jnp.where(ok, mean, 0.0)
            outv[...] = jnp.full((_LANES,), 1.0, jnp.float32) * res
            pltpu.sync_copy(outv, out_hbm)


def kernel(probs, targets):
    bits_all = jnp.asarray(_RAW_BITS)
    tgt = targets.astype(jnp.int32)
    mesh = plsc.VectorSubcoreMesh(core_axis_name="c", subcore_axis_name="s", num_cores=1)
    f = pl.kernel(
        _body,
        out_type=jax.ShapeDtypeStruct((_LANES,), jnp.float32),
        mesh=mesh,
        compiler_params=pltpu.CompilerParams(needs_layout_passes=False),
        scratch_types=[
            pltpu.VMEM((_CHUNK,), jnp.float32),
            pltpu.VMEM((_CHUNK,), jnp.int32),
            pltpu.VMEM((2 * _CHUNK,), jnp.float32),
            pltpu.VMEM((_LANES,), jnp.int32),
            pltpu.VMEM((4 * _SPT,), jnp.int32),
            pltpu.VMEM((_NT, _LANES), jnp.int32),
            pltpu.VMEM((_LANES,), jnp.int32),
            pltpu.VMEM((_LANES,), jnp.int32),
            pltpu.VMEM((_LANES,), jnp.int32),
            pltpu.VMEM((_LANES,), jnp.int32),
            pltpu.VMEM((2 * _SPT,), jnp.int32),
            pltpu.VMEM((2 * _N,), jnp.float32),
            pltpu.VMEM((_NT * 2 * _SPT,), jnp.int32),
            pltpu.VMEM((_LANES,), jnp.float32),
            pltpu.SemaphoreType.DMA,
            pltpu.SemaphoreType.DMA,
            pltpu.SemaphoreType.DMA,
            pltpu.VMEM_SHARED((2 * _N,), jnp.float32),
            pltpu.VMEM_SHARED((_NT, _LANES), jnp.int32),
            pltpu.VMEM_SHARED((_NT * 2 * _SPT,), jnp.int32),
        ],
    )
    out = f(probs, tgt, bits_all)
    return out[0]


# speedup vs baseline: 1.0911x; 1.0911x over previous
import jax
import jax.numpy as jnp
import numpy as np
from jax import lax
from jax.experimental import pallas as pl
from jax.experimental.pallas import tpu as pltpu
from jax.experimental.pallas import tpu_sc as plsc

_MARGIN = 0.1
_N = 16384
_PAIRS = 8192
_LANES = 16
_NT = 16
_CHUNK = _N // _NT
_SPT = _PAIRS // _NT


def _rotl32(x, r):
    return ((x << np.uint32(r)) | (x >> np.uint32(32 - r))).astype(np.uint32)


def _threefry2x32(k1, k2, x0, x1):
    x0 = x0.astype(np.uint32).copy()
    x1 = x1.astype(np.uint32).copy()
    ks = [np.uint32(k1), np.uint32(k2),
          np.uint32(np.uint32(k1) ^ np.uint32(k2) ^ np.uint32(0x1BD11BDA))]
    rotations = [[13, 15, 26, 6], [17, 29, 16, 24]]
    x0 += ks[0]
    x1 += ks[1]
    for i in range(5):
        for r in rotations[i % 2]:
            x0 += x1
            x1 = _rotl32(x1, r)
            x1 ^= x0
        x0 += ks[(i + 1) % 3]
        x1 += np.uint32(ks[(i + 2) % 3] + np.uint32(i + 1))
    return x0, x1


def _fry_bits(k, n):
    i = np.arange(n, dtype=np.uint64)
    o0, o1 = _threefry2x32(k[0], k[1], (i >> np.uint64(32)).astype(np.uint32),
                           (i & np.uint64(0xFFFFFFFF)).astype(np.uint32))
    return o0 ^ o1


def _fry_split(k):
    o0, o1 = _threefry2x32(k[0], k[1], np.zeros(2, np.uint32),
                           np.arange(2, dtype=np.uint32))
    return (o0[0], o1[0]), (o0[1], o1[1])


def _sample_bits():
    sk1, sk2 = _fry_split((np.uint32(0), np.uint32(42)))
    out = []
    for k in (sk1, sk2):
        ra, rb = _fry_split(k)
        for kk in (ra, rb):
            out.append(_fry_bits(kk, _PAIRS).view(np.int32))
    return tuple(out)


def _packed_bits():
    ue, ve, un, vn = _sample_bits()
    tiles = []
    for w in range(_NT):
        sl = slice(w * _SPT, (w + 1) * _SPT)
        tiles.append(np.concatenate([ue[sl], ve[sl], un[sl], vn[sl]]))
    return np.concatenate(tiles)


_RAW_BITS = _packed_bits()


def _vmod(x, span_v, rinv_v):
    q1 = (x.astype(jnp.float32) * rinv_v).astype(jnp.int32)
    r = x - q1 * span_v
    q2 = (r.astype(jnp.float32) * rinv_v).astype(jnp.int32)
    r = r - q2 * span_v
    r = jnp.where(r >= span_v, r - span_v, r)
    r = jnp.where(r < 0, r + span_v, r)
    r = jnp.where(r < 0, r + span_v, r)
    return r


def _ridx(u, v, bf, bh, bg, span_v, rinv_v):
    uhi = lax.shift_right_logical(u, 16)
    ulo = lax.bitwise_and(u, 0xFFFF)
    vhi = lax.shift_right_logical(v, 16)
    vlo = lax.bitwise_and(v, 0xFFFF)
    p1 = _vmod(uhi * bg + ulo * bh, span_v, rinv_v)
    r = _vmod(p1 + vhi * bf + vlo, span_v, rinv_v)
    return jnp.minimum(jnp.maximum(r, 0), span_v - 1)


def _chunk_of(a, ce_tab):
    w = a * 0
    for s in (8, 4, 2, 1):
        probe = plsc.load_gather(ce_tab, [w + (s - 1)])
        w = w + jnp.where(probe <= a, s, 0)
    return w


def _body(probs_hbm, tgt_hbm, bits_hbm, out_hbm,
          pv, tv, locb, cntw, bits, cnts, ce_tab, cx_tab, cen_tab, cxn_tab,
          pos_loc, buf, posf, accw, accl, outv, sem_bits, sem_buf, sem_in,
          sh_data, sh_cnt, sh_acc, sh_pos):
    cid = lax.axis_index("c")
    sid = lax.axis_index("s")

    @pl.when(cid == 0)
    def _():
        lane = lax.iota(jnp.int32, _LANES)

        cp_bits = pltpu.async_copy(bits_hbm.at[pl.ds(sid * 4 * _SPT, 4 * _SPT)],
                                   bits, sem_bits)

        base = sid * _CHUNK
        cp_p = pltpu.async_copy(probs_hbm.at[pl.ds(base, _CHUNK)], pv, sem_in)
        cp_t = pltpu.async_copy(tgt_hbm.at[pl.ds(base, _CHUNK)], tv, sem_in)
        cp_p.wait()
        cp_t.wait()

        def compact4(i, off_v):
            for k in range(8):
                j = i * 8 + k
                sl = pl.ds(j * _LANES, _LANES)
                t = tv[sl]
                p = pv[sl]
                m32 = jnp.where(t == 1, 1, 0)
                rank_e = plsc.cumsum(m32) - m32
                pc = plsc.all_reduce_population_count(t == 1)
                dest = jnp.where(m32 == 1, off_v + rank_e,
                                 (_CHUNK + j * _LANES) + lane - off_v - rank_e)
                plsc.store_scatter(locb, [dest], p)
                off_v = off_v + pc
            return off_v

        off_v = lax.fori_loop(0, _CHUNK // _LANES // 8, compact4,
                              jnp.zeros((_LANES,), jnp.int32))
        pltpu.sync_copy(locb, sh_data.at[pl.ds(sid * 2 * _CHUNK, 2 * _CHUNK)])
        cntw[...] = off_v
        pltpu.sync_copy(cntw, sh_cnt.at[sid])
        plsc.subcore_barrier()

        @pl.when(jnp.logical_or(sid == 0, sid == 8))
        def _():
            pltpu.async_copy(sh_data, buf, sem_buf)

        pltpu.sync_copy(sh_cnt, cnts)
        ce_vec = plsc.load_gather(cnts, [lane, lane * 0])
        ce_inc = plsc.cumsum(ce_vec)
        cx_exc = ce_inc - ce_vec
        cn_vec = _CHUNK - ce_vec
        cn_inc = plsc.cumsum(cn_vec)
        cxn_exc = cn_inc - cn_vec
        ce_tab[...] = ce_inc
        cx_tab[...] = cx_exc
        cen_tab[...] = cn_inc
        cxn_tab[...] = cxn_exc
        n_ess = ce_inc[_LANES - 1]
        n_non = _N - n_ess

        cp_bits.wait()

        se_v = lane * 0 + jnp.maximum(n_ess, 1)
        sn_v = lane * 0 + jnp.maximum(n_non, 1)
        rinv_e = 1.0 / se_v.astype(jnp.float32)
        rinv_n = 1.0 / sn_v.astype(jnp.float32)
        c64k = jnp.full((_LANES,), 65536, jnp.int32)
        bf_e = _vmod(c64k, se_v, rinv_e)
        bh_e = _vmod(bf_e * bf_e, se_v, rinv_e)
        bg_e = _vmod(lax.shift_left(bh_e, 16), se_v, rinv_e)
        bf_n = _vmod(c64k, sn_v, rinv_n)
        bh_n = _vmod(bf_n * bf_n, sn_v, rinv_n)
        bg_n = _vmod(lax.shift_left(bh_n, 16), sn_v, rinv_n)

        def samp4(i, carry):
            for k in range(4):
                v = i * 4 + k
                sl0 = pl.ds(v * _LANES, _LANES)
                a = _ridx(bits[sl0], bits[pl.ds(_SPT + v * _LANES, _LANES)],
                          bf_e, bh_e, bg_e, se_v, rinv_e)
                b = _ridx(bits[pl.ds(2 * _SPT + v * _LANES, _LANES)],
                          bits[pl.ds(3 * _SPT + v * _LANES, _LANES)],
                          bf_n, bh_n, bg_n, sn_v, rinv_n)
                wa = _chunk_of(a, ce_tab)
                pos_a = lax.shift_left(wa, 11) + a - plsc.load_gather(cx_tab, [wa])
                wb = _chunk_of(b, cen_tab)
                pos_b = (lax.shift_left(wb, 11) + _CHUNK + b
                         - plsc.load_gather(cxn_tab, [wb]))
                pos_loc[sl0] = pos_a
                pos_loc[pl.ds(_SPT + v * _LANES, _LANES)] = pos_b
            return carry

        lax.fori_loop(0, _SPT // _LANES // 4, samp4, jnp.int32(0))
        pltpu.sync_copy(pos_loc, sh_pos.at[pl.ds(sid * 2 * _SPT, 2 * _SPT)])
        plsc.subcore_barrier()

        @pl.when(jnp.logical_or(sid == 0, sid == 8))
        def _():
            rest = pl.ds((sid + 1) * 2 * _SPT, 7 * 2 * _SPT)
            cp_pos = pltpu.async_copy(sh_pos.at[rest], posf.at[rest], sem_in)
            pltpu.make_async_copy(sh_data, buf, sem_buf).wait()

            def mk_hinge8(src, rbase):
                def hinge8(i, a_):
                    for k in range(8):
                        o = (i * 8 + k) * _LANES
                        ra = src[pl.ds(rbase + o, _LANES)]
                        rb = src[pl.ds(rbase + _SPT + o, _LANES)]
                        pe = plsc.load_gather(buf, [ra])
                        pn = plsc.load_gather(buf, [rb])
                        a_ = a_ + jnp.maximum(pn - pe + _MARGIN, 0.0)
                    return a_
                return hinge8

            acc = lax.fori_loop(0, _SPT // _LANES // 8,
                                mk_hinge8(pos_loc, 0),
                                jnp.zeros((_LANES,), jnp.float32))
            cp_pos.wait()
            for w in range(1, 8):
                acc = lax.fori_loop(0, _SPT // _LANES // 8,
                                    mk_hinge8(posf, (sid + w) * 2 * _SPT), acc)
            accw[...] = acc
            pltpu.sync_copy(accw, sh_acc.at[sid])
        plsc.subcore_barrier()

        @pl.when(sid == 0)
        def _():
            pltpu.sync_copy(sh_acc, accl)
            p0 = plsc.load_gather(accl, [lane * 0, lane])
            p8 = plsc.load_gather(accl, [lane * 0 + 8, lane])
            mean = jnp.sum(p0 + p8) * (1.0 / _PAIRS)
            ok = jnp.logical_and(n_ess > 0, n_non > 0)
            res = jnp.where(ok, mean, 0.0)
            outv[...] = jnp.full((_LANES,), 1.0, jnp.float32) * res
            pltpu.sync_copy(outv, out_hbm)


def kernel(probs, targets):
    bits_all = jnp.asarray(_RAW_BITS)
    tgt = targets.astype(jnp.int32)
    mesh = plsc.VectorSubcoreMesh(core_axis_name="c", subcore_axis_name="s", num_cores=1)
    f = pl.kernel(
        _body,
        out_type=jax.ShapeDtypeStruct((_LANES,), jnp.float32),
        mesh=mesh,
        compiler_params=pltpu.CompilerParams(needs_layout_passes=False),
        scratch_types=[
            pltpu.VMEM((_CHUNK,), jnp.float32),
            pltpu.VMEM((_CHUNK,), jnp.int32),
            pltpu.VMEM((2 * _CHUNK,), jnp.float32),
            pltpu.VMEM((_LANES,), jnp.int32),
            pltpu.VMEM((4 * _SPT,), jnp.int32),
            pltpu.VMEM((_NT, _LANES), jnp.int32),
            pltpu.VMEM((_LANES,), jnp.int32),
            pltpu.VMEM((_LANES,), jnp.int32),
            pltpu.VMEM((_LANES,), jnp.int32),
            pltpu.VMEM((_LANES,), jnp.int32),
            pltpu.VMEM((2 * _SPT,), jnp.int32),
            pltpu.VMEM((2 * _N,), jnp.float32),
            pltpu.VMEM((_NT * 2 * _SPT,), jnp.int32),
            pltpu.VMEM((_LANES,), jnp.float32),
            pltpu.VMEM((_NT, _LANES), jnp.float32),
            pltpu.VMEM((_LANES,), jnp.float32),
            pltpu.SemaphoreType.DMA,
            pltpu.SemaphoreType.DMA,
            pltpu.SemaphoreType.DMA,
            pltpu.VMEM_SHARED((2 * _N,), jnp.float32),
            pltpu.VMEM_SHARED((_NT, _LANES), jnp.int32),
            pltpu.VMEM_SHARED((_NT, _LANES), jnp.float32),
            pltpu.VMEM_SHARED((_NT * 2 * _SPT,), jnp.int32),
        ],
    )
    out = f(probs, tgt, bits_all)
    return out[0]


# speedup vs baseline: 1.1561x; 1.0596x over previous
import jax
import jax.numpy as jnp
import numpy as np
from jax import lax
from jax.experimental import pallas as pl
from jax.experimental.pallas import tpu as pltpu
from jax.experimental.pallas import tpu_sc as plsc

_MARGIN = 0.1
_N = 16384
_PAIRS = 8192
_LANES = 16
_NT = 16
_CHUNK = _N // _NT
_SPT = _PAIRS // _NT


def _rotl32(x, r):
    return ((x << np.uint32(r)) | (x >> np.uint32(32 - r))).astype(np.uint32)


def _threefry2x32(k1, k2, x0, x1):
    x0 = x0.astype(np.uint32).copy()
    x1 = x1.astype(np.uint32).copy()
    ks = [np.uint32(k1), np.uint32(k2),
          np.uint32(np.uint32(k1) ^ np.uint32(k2) ^ np.uint32(0x1BD11BDA))]
    rotations = [[13, 15, 26, 6], [17, 29, 16, 24]]
    x0 += ks[0]
    x1 += ks[1]
    for i in range(5):
        for r in rotations[i % 2]:
            x0 += x1
            x1 = _rotl32(x1, r)
            x1 ^= x0
        x0 += ks[(i + 1) % 3]
        x1 += np.uint32(ks[(i + 2) % 3] + np.uint32(i + 1))
    return x0, x1


def _fry_bits(k, n):
    i = np.arange(n, dtype=np.uint64)
    o0, o1 = _threefry2x32(k[0], k[1], (i >> np.uint64(32)).astype(np.uint32),
                           (i & np.uint64(0xFFFFFFFF)).astype(np.uint32))
    return o0 ^ o1


def _fry_split(k):
    o0, o1 = _threefry2x32(k[0], k[1], np.zeros(2, np.uint32),
                           np.arange(2, dtype=np.uint32))
    return (o0[0], o1[0]), (o0[1], o1[1])


def _sample_bits():
    sk1, sk2 = _fry_split((np.uint32(0), np.uint32(42)))
    out = []
    for k in (sk1, sk2):
        ra, rb = _fry_split(k)
        for kk in (ra, rb):
            out.append(_fry_bits(kk, _PAIRS).view(np.int32))
    return tuple(out)


def _packed_bits():
    ue, ve, un, vn = _sample_bits()
    tiles = []
    for w in range(_NT):
        sl = slice(w * _SPT, (w + 1) * _SPT)
        tiles.append(np.concatenate([ue[sl], ve[sl], un[sl], vn[sl]]))
    return np.concatenate(tiles)


_RAW_BITS = _packed_bits()


def _vmod(x, span_v, rinv_v):
    q1 = (x.astype(jnp.float32) * rinv_v).astype(jnp.int32)
    r = x - q1 * span_v
    q2 = (r.astype(jnp.float32) * rinv_v).astype(jnp.int32)
    r = r - q2 * span_v
    r = jnp.where(r >= span_v, r - span_v, r)
    r = jnp.where(r < 0, r + span_v, r)
    r = jnp.where(r < 0, r + span_v, r)
    return r


def _ridx(u, v, bf, bh, bg, span_v, rinv_v):
    uhi = lax.shift_right_logical(u, 16)
    ulo = lax.bitwise_and(u, 0xFFFF)
    vhi = lax.shift_right_logical(v, 16)
    vlo = lax.bitwise_and(v, 0xFFFF)
    p1 = _vmod(uhi * bg + ulo * bh, span_v, rinv_v)
    r = _vmod(p1 + vhi * bf + vlo, span_v, rinv_v)
    return jnp.minimum(jnp.maximum(r, 0), span_v - 1)


def _chunk_of(a, ce_tab):
    w = a * 0
    for s in (8, 4, 2, 1):
        probe = plsc.load_gather(ce_tab, [w + (s - 1)])
        w = w + jnp.where(probe <= a, s, 0)
    return w


def _body(probs_hbm, tgt_hbm, bits_hbm, out_hbm,
          pv, tv, locb, cntw, bits, cnts, ce_tab, cx_tab, cen_tab, cxn_tab,
          pos_loc, buf, posf, accw, accl, outv, sem_bits, sem_buf, sem_in,
          sh_data, sh_cnt, sh_acc, sh_pos):
    cid = lax.axis_index("c")
    sid = lax.axis_index("s")

    @pl.when(cid == 0)
    def _():
        lane = lax.iota(jnp.int32, _LANES)

        cp_bits = pltpu.async_copy(bits_hbm.at[pl.ds(sid * 4 * _SPT, 4 * _SPT)],
                                   bits, sem_bits)

        base = sid * _CHUNK
        cp_p = pltpu.async_copy(probs_hbm.at[pl.ds(base, _CHUNK)], pv, sem_in)
        cp_t = pltpu.async_copy(tgt_hbm.at[pl.ds(base, _CHUNK)], tv, sem_in)
        cp_p.wait()
        cp_t.wait()

        def compact4(i, off_v):
            for k in range(8):
                j = i * 8 + k
                sl = pl.ds(j * _LANES, _LANES)
                t = tv[sl]
                p = pv[sl]
                m32 = jnp.where(t == 1, 1, 0)
                rank_e = plsc.cumsum(m32) - m32
                pc = plsc.all_reduce_population_count(t == 1)
                dest = jnp.where(m32 == 1, off_v + rank_e,
                                 (_CHUNK + j * _LANES) + lane - off_v - rank_e)
                plsc.store_scatter(locb, [dest], p)
                off_v = off_v + pc
            return off_v

        off_v = lax.fori_loop(0, _CHUNK // _LANES // 8, compact4,
                              jnp.zeros((_LANES,), jnp.int32))
        pltpu.sync_copy(locb, sh_data.at[pl.ds(sid * 2 * _CHUNK, 2 * _CHUNK)])
        cntw[...] = off_v
        pltpu.sync_copy(cntw, sh_cnt.at[sid])
        plsc.subcore_barrier()

        @pl.when(lax.rem(sid, 4) == 0)
        def _():
            pltpu.async_copy(sh_data, buf, sem_buf)

        pltpu.sync_copy(sh_cnt, cnts)
        ce_vec = plsc.load_gather(cnts, [lane, lane * 0])
        ce_inc = plsc.cumsum(ce_vec)
        cx_exc = ce_inc - ce_vec
        cn_vec = _CHUNK - ce_vec
        cn_inc = plsc.cumsum(cn_vec)
        cxn_exc = cn_inc - cn_vec
        ce_tab[...] = ce_inc
        cx_tab[...] = cx_exc
        cen_tab[...] = cn_inc
        cxn_tab[...] = cxn_exc
        n_ess = ce_inc[_LANES - 1]
        n_non = _N - n_ess

        cp_bits.wait()

        se_v = lane * 0 + jnp.maximum(n_ess, 1)
        sn_v = lane * 0 + jnp.maximum(n_non, 1)
        rinv_e = 1.0 / se_v.astype(jnp.float32)
        rinv_n = 1.0 / sn_v.astype(jnp.float32)
        c64k = jnp.full((_LANES,), 65536, jnp.int32)
        bf_e = _vmod(c64k, se_v, rinv_e)
        bh_e = _vmod(bf_e * bf_e, se_v, rinv_e)
        bg_e = _vmod(lax.shift_left(bh_e, 16), se_v, rinv_e)
        bf_n = _vmod(c64k, sn_v, rinv_n)
        bh_n = _vmod(bf_n * bf_n, sn_v, rinv_n)
        bg_n = _vmod(lax.shift_left(bh_n, 16), sn_v, rinv_n)

        def samp4(i, carry):
            for k in range(4):
                v = i * 4 + k
                sl0 = pl.ds(v * _LANES, _LANES)
                a = _ridx(bits[sl0], bits[pl.ds(_SPT + v * _LANES, _LANES)],
                          bf_e, bh_e, bg_e, se_v, rinv_e)
                b = _ridx(bits[pl.ds(2 * _SPT + v * _LANES, _LANES)],
                          bits[pl.ds(3 * _SPT + v * _LANES, _LANES)],
                          bf_n, bh_n, bg_n, sn_v, rinv_n)
                wa = _chunk_of(a, ce_tab)
                pos_a = lax.shift_left(wa, 11) + a - plsc.load_gather(cx_tab, [wa])
                wb = _chunk_of(b, cen_tab)
                pos_b = (lax.shift_left(wb, 11) + _CHUNK + b
                         - plsc.load_gather(cxn_tab, [wb]))
                pos_loc[sl0] = pos_a
                pos_loc[pl.ds(_SPT + v * _LANES, _LANES)] = pos_b
            return carry

        lax.fori_loop(0, _SPT // _LANES // 4, samp4, jnp.int32(0))
        pltpu.sync_copy(pos_loc, sh_pos.at[pl.ds(sid * 2 * _SPT, 2 * _SPT)])
        plsc.subcore_barrier()

        @pl.when(lax.rem(sid, 4) == 0)
        def _():
            rest = pl.ds((sid + 1) * 2 * _SPT, 3 * 2 * _SPT)
            cp_pos = pltpu.async_copy(sh_pos.at[rest], posf.at[rest], sem_in)
            pltpu.make_async_copy(sh_data, buf, sem_buf).wait()

            def mk_hinge8(src, rbase):
                def hinge8(i, a_):
                    for k in range(8):
                        o = (i * 8 + k) * _LANES
                        ra = src[pl.ds(rbase + o, _LANES)]
                        rb = src[pl.ds(rbase + _SPT + o, _LANES)]
                        pe = plsc.load_gather(buf, [ra])
                        pn = plsc.load_gather(buf, [rb])
                        a_ = a_ + jnp.maximum(pn - pe + _MARGIN, 0.0)
                    return a_
                return hinge8

            acc = lax.fori_loop(0, _SPT // _LANES // 8,
                                mk_hinge8(pos_loc, 0),
                                jnp.zeros((_LANES,), jnp.float32))
            cp_pos.wait()
            for w in range(1, 4):
                acc = lax.fori_loop(0, _SPT // _LANES // 8,
                                    mk_hinge8(posf, (sid + w) * 2 * _SPT), acc)
            accw[...] = acc
            pltpu.sync_copy(accw, sh_acc.at[sid])
        plsc.subcore_barrier()

        @pl.when(sid == 0)
        def _():
            pltpu.sync_copy(sh_acc, accl)
            tot = jnp.zeros((_LANES,), jnp.float32)
            for r in (0, 4, 8, 12):
                tot = tot + plsc.load_gather(accl, [lane * 0 + r, lane])
            mean = jnp.sum(tot) * (1.0 / _PAIRS)
            ok = jnp.logical_and(n_ess > 0, n_non > 0)
            res = jnp.where(ok, mean, 0.0)
            outv[...] = jnp.full((_LANES,), 1.0, jnp.float32) * res
            pltpu.sync_copy(outv, out_hbm)


def kernel(probs, targets):
    bits_all = jnp.asarray(_RAW_BITS)
    tgt = targets.astype(jnp.int32)
    mesh = plsc.VectorSubcoreMesh(core_axis_name="c", subcore_axis_name="s", num_cores=1)
    f = pl.kernel(
        _body,
        out_type=jax.ShapeDtypeStruct((_LANES,), jnp.float32),
        mesh=mesh,
        compiler_params=pltpu.CompilerParams(needs_layout_passes=False),
        scratch_types=[
            pltpu.VMEM((_CHUNK,), jnp.float32),
            pltpu.VMEM((_CHUNK,), jnp.int32),
            pltpu.VMEM((2 * _CHUNK,), jnp.float32),
            pltpu.VMEM((_LANES,), jnp.int32),
            pltpu.VMEM((4 * _SPT,), jnp.int32),
            pltpu.VMEM((_NT, _LANES), jnp.int32),
            pltpu.VMEM((_LANES,), jnp.int32),
            pltpu.VMEM((_LANES,), jnp.int32),
            pltpu.VMEM((_LANES,), jnp.int32),
            pltpu.VMEM((_LANES,), jnp.int32),
            pltpu.VMEM((2 * _SPT,), jnp.int32),
            pltpu.VMEM((2 * _N,), jnp.float32),
            pltpu.VMEM((_NT * 2 * _SPT,), jnp.int32),
            pltpu.VMEM((_LANES,), jnp.float32),
            pltpu.VMEM((_NT, _LANES), jnp.float32),
            pltpu.VMEM((_LANES,), jnp.float32),
            pltpu.SemaphoreType.DMA,
            pltpu.SemaphoreType.DMA,
            pltpu.SemaphoreType.DMA,
            pltpu.VMEM_SHARED((2 * _N,), jnp.float32),
            pltpu.VMEM_SHARED((_NT, _LANES), jnp.int32),
            pltpu.VMEM_SHARED((_NT, _LANES), jnp.float32),
            pltpu.VMEM_SHARED((_NT * 2 * _SPT,), jnp.int32),
        ],
    )
    out = f(probs, tgt, bits_all)
    return out[0]


# speedup vs baseline: 1.1804x; 1.0210x over previous
import jax
import jax.numpy as jnp
import numpy as np
from jax import lax
from jax.experimental import pallas as pl
from jax.experimental.pallas import tpu as pltpu
from jax.experimental.pallas import tpu_sc as plsc

_MARGIN = 0.1
_N = 16384
_PAIRS = 8192
_LANES = 16
_NT = 16
_CHUNK = _N // _NT
_SPT = _PAIRS // _NT


def _rotl32(x, r):
    return ((x << np.uint32(r)) | (x >> np.uint32(32 - r))).astype(np.uint32)


def _threefry2x32(k1, k2, x0, x1):
    x0 = x0.astype(np.uint32).copy()
    x1 = x1.astype(np.uint32).copy()
    ks = [np.uint32(k1), np.uint32(k2),
          np.uint32(np.uint32(k1) ^ np.uint32(k2) ^ np.uint32(0x1BD11BDA))]
    rotations = [[13, 15, 26, 6], [17, 29, 16, 24]]
    x0 += ks[0]
    x1 += ks[1]
    for i in range(5):
        for r in rotations[i % 2]:
            x0 += x1
            x1 = _rotl32(x1, r)
            x1 ^= x0
        x0 += ks[(i + 1) % 3]
        x1 += np.uint32(ks[(i + 2) % 3] + np.uint32(i + 1))
    return x0, x1


def _fry_bits(k, n):
    i = np.arange(n, dtype=np.uint64)
    o0, o1 = _threefry2x32(k[0], k[1], (i >> np.uint64(32)).astype(np.uint32),
                           (i & np.uint64(0xFFFFFFFF)).astype(np.uint32))
    return o0 ^ o1


def _fry_split(k):
    o0, o1 = _threefry2x32(k[0], k[1], np.zeros(2, np.uint32),
                           np.arange(2, dtype=np.uint32))
    return (o0[0], o1[0]), (o0[1], o1[1])


def _sample_bits():
    sk1, sk2 = _fry_split((np.uint32(0), np.uint32(42)))
    out = []
    for k in (sk1, sk2):
        ra, rb = _fry_split(k)
        for kk in (ra, rb):
            out.append(_fry_bits(kk, _PAIRS).view(np.int32))
    return tuple(out)


def _packed_bits():
    ue, ve, un, vn = _sample_bits()
    tiles = []
    for w in range(_NT):
        sl = slice(w * _SPT, (w + 1) * _SPT)
        tiles.append(np.concatenate([ue[sl], ve[sl], un[sl], vn[sl]]))
    return np.concatenate(tiles)


_RAW_BITS = _packed_bits()


def _vmod(x, span_v, rinv_v):
    q1 = (x.astype(jnp.float32) * rinv_v).astype(jnp.int32)
    r = x - q1 * span_v
    q2 = (r.astype(jnp.float32) * rinv_v).astype(jnp.int32)
    r = r - q2 * span_v
    r = jnp.where(r >= span_v, r - span_v, r)
    r = jnp.where(r < 0, r + span_v, r)
    r = jnp.where(r < 0, r + span_v, r)
    return r


def _ridx(u, v, bf, bh, bg, span_v, rinv_v):
    uhi = lax.shift_right_logical(u, 16)
    ulo = lax.bitwise_and(u, 0xFFFF)
    vhi = lax.shift_right_logical(v, 16)
    vlo = lax.bitwise_and(v, 0xFFFF)
    p1 = _vmod(uhi * bg + ulo * bh, span_v, rinv_v)
    r = _vmod(p1 + vhi * bf + vlo, span_v, rinv_v)
    return jnp.minimum(jnp.maximum(r, 0), span_v - 1)


def _chunk_of(a, ce_tab):
    w = a * 0
    for s in (8, 4, 2, 1):
        probe = plsc.load_gather(ce_tab, [w + (s - 1)])
        w = w + jnp.where(probe <= a, s, 0)
    return w


def _body(probs_hbm, tgt_hbm, bits_hbm, out_hbm,
          pv, tv, locb, cntw, bits, cnts, ce_tab, cx_tab, cen_tab, cxn_tab,
          pos_loc, buf, posf, accw, accl, outv, sem_bits, sem_buf, sem_in,
          sh_data, sh_cnt, sh_acc, sh_pos):
    cid = lax.axis_index("c")
    sid = lax.axis_index("s")

    @pl.when(cid == 0)
    def _():
        lane = lax.iota(jnp.int32, _LANES)

        cp_bits = pltpu.async_copy(bits_hbm.at[pl.ds(sid * 4 * _SPT, 4 * _SPT)],
                                   bits, sem_bits)

        base = sid * _CHUNK
        cp_p = pltpu.async_copy(probs_hbm.at[pl.ds(base, _CHUNK)], pv, sem_in)
        cp_t = pltpu.async_copy(tgt_hbm.at[pl.ds(base, _CHUNK)], tv, sem_in)
        cp_p.wait()
        cp_t.wait()

        def compact4(i, off_v):
            for k in range(8):
                j = i * 8 + k
                sl = pl.ds(j * _LANES, _LANES)
                t = tv[sl]
                p = pv[sl]
                m32 = jnp.where(t == 1, 1, 0)
                rank_e = plsc.cumsum(m32) - m32
                pc = plsc.all_reduce_population_count(t == 1)
                dest = jnp.where(m32 == 1, off_v + rank_e,
                                 (_CHUNK + j * _LANES) + lane - off_v - rank_e)
                plsc.store_scatter(locb, [dest], p)
                off_v = off_v + pc
            return off_v

        off_v = lax.fori_loop(0, _CHUNK // _LANES // 8, compact4,
                              jnp.zeros((_LANES,), jnp.int32))
        pltpu.sync_copy(locb, sh_data.at[pl.ds(sid * 2 * _CHUNK, 2 * _CHUNK)])
        cntw[...] = off_v
        pltpu.sync_copy(cntw, sh_cnt.at[sid])
        plsc.subcore_barrier()

        @pl.when(lax.rem(sid, 2) == 0)
        def _():
            pltpu.async_copy(sh_data, buf, sem_buf)

        pltpu.sync_copy(sh_cnt, cnts)
        ce_vec = plsc.load_gather(cnts, [lane, lane * 0])
        ce_inc = plsc.cumsum(ce_vec)
        cx_exc = ce_inc - ce_vec
        cn_vec = _CHUNK - ce_vec
        cn_inc = plsc.cumsum(cn_vec)
        cxn_exc = cn_inc - cn_vec
        ce_tab[...] = ce_inc
        cx_tab[...] = cx_exc
        cen_tab[...] = cn_inc
        cxn_tab[...] = cxn_exc
        n_ess = ce_inc[_LANES - 1]
        n_non = _N - n_ess

        cp_bits.wait()

        se_v = lane * 0 + jnp.maximum(n_ess, 1)
        sn_v = lane * 0 + jnp.maximum(n_non, 1)
        rinv_e = 1.0 / se_v.astype(jnp.float32)
        rinv_n = 1.0 / sn_v.astype(jnp.float32)
        c64k = jnp.full((_LANES,), 65536, jnp.int32)
        bf_e = _vmod(c64k, se_v, rinv_e)
        bh_e = _vmod(bf_e * bf_e, se_v, rinv_e)
        bg_e = _vmod(lax.shift_left(bh_e, 16), se_v, rinv_e)
        bf_n = _vmod(c64k, sn_v, rinv_n)
        bh_n = _vmod(bf_n * bf_n, sn_v, rinv_n)
        bg_n = _vmod(lax.shift_left(bh_n, 16), sn_v, rinv_n)

        def samp4(i, carry):
            for k in range(4):
                v = i * 4 + k
                sl0 = pl.ds(v * _LANES, _LANES)
                a = _ridx(bits[sl0], bits[pl.ds(_SPT + v * _LANES, _LANES)],
                          bf_e, bh_e, bg_e, se_v, rinv_e)
                b = _ridx(bits[pl.ds(2 * _SPT + v * _LANES, _LANES)],
                          bits[pl.ds(3 * _SPT + v * _LANES, _LANES)],
                          bf_n, bh_n, bg_n, sn_v, rinv_n)
                wa = _chunk_of(a, ce_tab)
                pos_a = lax.shift_left(wa, 11) + a - plsc.load_gather(cx_tab, [wa])
                wb = _chunk_of(b, cen_tab)
                pos_b = (lax.shift_left(wb, 11) + _CHUNK + b
                         - plsc.load_gather(cxn_tab, [wb]))
                pos_loc[sl0] = pos_a
                pos_loc[pl.ds(_SPT + v * _LANES, _LANES)] = pos_b
            return carry

        lax.fori_loop(0, _SPT // _LANES // 4, samp4, jnp.int32(0))
        pltpu.sync_copy(pos_loc, sh_pos.at[pl.ds(sid * 2 * _SPT, 2 * _SPT)])
        plsc.subcore_barrier()

        @pl.when(lax.rem(sid, 2) == 0)
        def _():
            rest = pl.ds((sid + 1) * 2 * _SPT, 2 * _SPT)
            cp_pos = pltpu.async_copy(sh_pos.at[rest], posf.at[rest], sem_in)
            pltpu.make_async_copy(sh_data, buf, sem_buf).wait()

            def mk_hinge8(src, rbase):
                def hinge8(i, a_):
                    for k in range(8):
                        o = (i * 8 + k) * _LANES
                        ra = src[pl.ds(rbase + o, _LANES)]
                        rb = src[pl.ds(rbase + _SPT + o, _LANES)]
                        pe = plsc.load_gather(buf, [ra])
                        pn = plsc.load_gather(buf, [rb])
                        a_ = a_ + jnp.maximum(pn - pe + _MARGIN, 0.0)
                    return a_
                return hinge8

            acc = lax.fori_loop(0, _SPT // _LANES // 8,
                                mk_hinge8(pos_loc, 0),
                                jnp.zeros((_LANES,), jnp.float32))
            cp_pos.wait()
            for w in range(1, 2):
                acc = lax.fori_loop(0, _SPT // _LANES // 8,
                                    mk_hinge8(posf, (sid + w) * 2 * _SPT), acc)
            accw[...] = acc
            pltpu.sync_copy(accw, sh_acc.at[sid])
        plsc.subcore_barrier()

        @pl.when(sid == 0)
        def _():
            pltpu.sync_copy(sh_acc, accl)
            tot = jnp.zeros((_LANES,), jnp.float32)
            for r in (0, 2, 4, 6, 8, 10, 12, 14):
                tot = tot + plsc.load_gather(accl, [lane * 0 + r, lane])
            mean = jnp.sum(tot) * (1.0 / _PAIRS)
            ok = jnp.logical_and(n_ess > 0, n_non > 0)
            res = jnp.where(ok, mean, 0.0)
            outv[...] = jnp.full((_LANES,), 1.0, jnp.float32) * res
            pltpu.sync_copy(outv, out_hbm)


def kernel(probs, targets):
    bits_all = jnp.asarray(_RAW_BITS)
    tgt = targets.astype(jnp.int32)
    mesh = plsc.VectorSubcoreMesh(core_axis_name="c", subcore_axis_name="s", num_cores=1)
    f = pl.kernel(
        _body,
        out_type=jax.ShapeDtypeStruct((_LANES,), jnp.float32),
        mesh=mesh,
        compiler_params=pltpu.CompilerParams(needs_layout_passes=False),
        scratch_types=[
            pltpu.VMEM((_CHUNK,), jnp.float32),
            pltpu.VMEM((_CHUNK,), jnp.int32),
            pltpu.VMEM((2 * _CHUNK,), jnp.float32),
            pltpu.VMEM((_LANES,), jnp.int32),
            pltpu.VMEM((4 * _SPT,), jnp.int32),
            pltpu.VMEM((_NT, _LANES), jnp.int32),
            pltpu.VMEM((_LANES,), jnp.int32),
            pltpu.VMEM((_LANES,), jnp.int32),
            pltpu.VMEM((_LANES,), jnp.int32),
            pltpu.VMEM((_LANES,), jnp.int32),
            pltpu.VMEM((2 * _SPT,), jnp.int32),
            pltpu.VMEM((2 * _N,), jnp.float32),
            pltpu.VMEM((_NT * 2 * _SPT,), jnp.int32),
            pltpu.VMEM((_LANES,), jnp.float32),
            pltpu.VMEM((_NT, _LANES), jnp.float32),
            pltpu.VMEM((_LANES,), jnp.float32),
            pltpu.SemaphoreType.DMA,
            pltpu.SemaphoreType.DMA,
            pltpu.SemaphoreType.DMA,
            pltpu.VMEM_SHARED((2 * _N,), jnp.float32),
            pltpu.VMEM_SHARED((_NT, _LANES), jnp.int32),
            pltpu.VMEM_SHARED((_NT, _LANES), jnp.float32),
            pltpu.VMEM_SHARED((_NT * 2 * _SPT,), jnp.int32),
        ],
    )
    out = f(probs, tgt, bits_all)
    return out[0]


# speedup vs baseline: 1.1951x; 1.0125x over previous
import jax
import jax.numpy as jnp
import numpy as np
from jax import lax
from jax.experimental import pallas as pl
from jax.experimental.pallas import tpu as pltpu
from jax.experimental.pallas import tpu_sc as plsc

_MARGIN = 0.1
_N = 16384
_PAIRS = 8192
_LANES = 16
_NT = 16
_CHUNK = _N // _NT
_SPT = _PAIRS // _NT


def _rotl32(x, r):
    return ((x << np.uint32(r)) | (x >> np.uint32(32 - r))).astype(np.uint32)


def _threefry2x32(k1, k2, x0, x1):
    x0 = x0.astype(np.uint32).copy()
    x1 = x1.astype(np.uint32).copy()
    ks = [np.uint32(k1), np.uint32(k2),
          np.uint32(np.uint32(k1) ^ np.uint32(k2) ^ np.uint32(0x1BD11BDA))]
    rotations = [[13, 15, 26, 6], [17, 29, 16, 24]]
    x0 += ks[0]
    x1 += ks[1]
    for i in range(5):
        for r in rotations[i % 2]:
            x0 += x1
            x1 = _rotl32(x1, r)
            x1 ^= x0
        x0 += ks[(i + 1) % 3]
        x1 += np.uint32(ks[(i + 2) % 3] + np.uint32(i + 1))
    return x0, x1


def _fry_bits(k, n):
    i = np.arange(n, dtype=np.uint64)
    o0, o1 = _threefry2x32(k[0], k[1], (i >> np.uint64(32)).astype(np.uint32),
                           (i & np.uint64(0xFFFFFFFF)).astype(np.uint32))
    return o0 ^ o1


def _fry_split(k):
    o0, o1 = _threefry2x32(k[0], k[1], np.zeros(2, np.uint32),
                           np.arange(2, dtype=np.uint32))
    return (o0[0], o1[0]), (o0[1], o1[1])


def _sample_bits():
    sk1, sk2 = _fry_split((np.uint32(0), np.uint32(42)))
    out = []
    for k in (sk1, sk2):
        ra, rb = _fry_split(k)
        for kk in (ra, rb):
            out.append(_fry_bits(kk, _PAIRS).view(np.int32))
    return tuple(out)


def _packed_bits():
    ue, ve, un, vn = _sample_bits()
    tiles = []
    for w in range(_NT):
        sl = slice(w * _SPT, (w + 1) * _SPT)
        tiles.append(np.concatenate([ue[sl], ve[sl], un[sl], vn[sl]]))
    return np.concatenate(tiles)


_RAW_BITS = _packed_bits()


def _vmod(x, span_v, rinv_v):
    q1 = (x.astype(jnp.float32) * rinv_v).astype(jnp.int32)
    r = x - q1 * span_v
    q2 = (r.astype(jnp.float32) * rinv_v).astype(jnp.int32)
    r = r - q2 * span_v
    r = jnp.where(r >= span_v, r - span_v, r)
    r = jnp.where(r < 0, r + span_v, r)
    r = jnp.where(r < 0, r + span_v, r)
    return r


def _ridx(u, v, bf, bh, bg, span_v, rinv_v):
    uhi = lax.shift_right_logical(u, 16)
    ulo = lax.bitwise_and(u, 0xFFFF)
    vhi = lax.shift_right_logical(v, 16)
    vlo = lax.bitwise_and(v, 0xFFFF)
    p1 = _vmod(uhi * bg + ulo * bh, span_v, rinv_v)
    r = _vmod(p1 + vhi * bf + vlo, span_v, rinv_v)
    return jnp.minimum(jnp.maximum(r, 0), span_v - 1)


def _chunk_of(a, ce_tab):
    w = a * 0
    for s in (8, 4, 2, 1):
        probe = plsc.load_gather(ce_tab, [w + (s - 1)])
        w = w + jnp.where(probe <= a, s, 0)
    return w


def _body(probs_hbm, tgt_hbm, bits_hbm, out_hbm,
          pv, tv, locb, cntw, bits, cnts, ce_tab, cx_tab, cen_tab, cxn_tab,
          pos_loc, buf, posf, accw, accl, outv, sem_bits, sem_buf, sem_in,
          sh_data, sh_cnt, sh_acc, sh_pos):
    cid = lax.axis_index("c")
    sid = lax.axis_index("s")

    @pl.when(cid == 0)
    def _():
        lane = lax.iota(jnp.int32, _LANES)

        cp_bits = pltpu.async_copy(bits_hbm.at[pl.ds(sid * 4 * _SPT, 4 * _SPT)],
                                   bits, sem_bits)

        base = sid * _CHUNK
        cp_p = pltpu.async_copy(probs_hbm.at[pl.ds(base, _CHUNK)], pv, sem_in)
        cp_t = pltpu.async_copy(tgt_hbm.at[pl.ds(base, _CHUNK)], tv, sem_in)
        cp_p.wait()
        cp_t.wait()

        def compact4(i, off_v):
            for k in range(8):
                j = i * 8 + k
                sl = pl.ds(j * _LANES, _LANES)
                t = tv[sl]
                p = pv[sl]
                m32 = jnp.where(t == 1, 1, 0)
                rank_e = plsc.cumsum(m32) - m32
                pc = plsc.all_reduce_population_count(t == 1)
                dest = jnp.where(m32 == 1, off_v + rank_e,
                                 (_CHUNK + j * _LANES) + lane - off_v - rank_e)
                plsc.store_scatter(locb, [dest], p)
                off_v = off_v + pc
            return off_v

        off_v = lax.fori_loop(0, _CHUNK // _LANES // 8, compact4,
                              jnp.zeros((_LANES,), jnp.int32))
        pltpu.sync_copy(locb, sh_data.at[pl.ds(sid * 2 * _CHUNK, 2 * _CHUNK)])
        cntw[...] = off_v
        pltpu.sync_copy(cntw, sh_cnt.at[sid])
        plsc.subcore_barrier()

        pltpu.async_copy(sh_data, buf, sem_buf)

        pltpu.sync_copy(sh_cnt, cnts)
        ce_vec = plsc.load_gather(cnts, [lane, lane * 0])
        ce_inc = plsc.cumsum(ce_vec)
        cx_exc = ce_inc - ce_vec
        cn_vec = _CHUNK - ce_vec
        cn_inc = plsc.cumsum(cn_vec)
        cxn_exc = cn_inc - cn_vec
        ce_tab[...] = ce_inc
        cx_tab[...] = cx_exc
        cen_tab[...] = cn_inc
        cxn_tab[...] = cxn_exc
        n_ess = ce_inc[_LANES - 1]
        n_non = _N - n_ess

        cp_bits.wait()

        se_v = lane * 0 + jnp.maximum(n_ess, 1)
        sn_v = lane * 0 + jnp.maximum(n_non, 1)
        rinv_e = 1.0 / se_v.astype(jnp.float32)
        rinv_n = 1.0 / sn_v.astype(jnp.float32)
        c64k = jnp.full((_LANES,), 65536, jnp.int32)
        bf_e = _vmod(c64k, se_v, rinv_e)
        bh_e = _vmod(bf_e * bf_e, se_v, rinv_e)
        bg_e = _vmod(lax.shift_left(bh_e, 16), se_v, rinv_e)
        bf_n = _vmod(c64k, sn_v, rinv_n)
        bh_n = _vmod(bf_n * bf_n, sn_v, rinv_n)
        bg_n = _vmod(lax.shift_left(bh_n, 16), sn_v, rinv_n)

        def samp4(i, carry):
            for k in range(4):
                v = i * 4 + k
                sl0 = pl.ds(v * _LANES, _LANES)
                a = _ridx(bits[sl0], bits[pl.ds(_SPT + v * _LANES, _LANES)],
                          bf_e, bh_e, bg_e, se_v, rinv_e)
                b = _ridx(bits[pl.ds(2 * _SPT + v * _LANES, _LANES)],
                          bits[pl.ds(3 * _SPT + v * _LANES, _LANES)],
                          bf_n, bh_n, bg_n, sn_v, rinv_n)
                wa = _chunk_of(a, ce_tab)
                pos_a = lax.shift_left(wa, 11) + a - plsc.load_gather(cx_tab, [wa])
                wb = _chunk_of(b, cen_tab)
                pos_b = (lax.shift_left(wb, 11) + _CHUNK + b
                         - plsc.load_gather(cxn_tab, [wb]))
                pos_loc[sl0] = pos_a
                pos_loc[pl.ds(_SPT + v * _LANES, _LANES)] = pos_b
            return carry

        lax.fori_loop(0, _SPT // _LANES // 4, samp4, jnp.int32(0))
        pltpu.sync_copy(pos_loc, sh_pos.at[pl.ds(sid * 2 * _SPT, 2 * _SPT)])
        plsc.subcore_barrier()

        pltpu.make_async_copy(sh_data, buf, sem_buf).wait()

        def hinge8(i, a_):
            for k in range(8):
                o = (i * 8 + k) * _LANES
                ra = pos_loc[pl.ds(o, _LANES)]
                rb = pos_loc[pl.ds(_SPT + o, _LANES)]
                pe = plsc.load_gather(buf, [ra])
                pn = plsc.load_gather(buf, [rb])
                a_ = a_ + jnp.maximum(pn - pe + _MARGIN, 0.0)
            return a_

        acc = lax.fori_loop(0, _SPT // _LANES // 8, hinge8,
                            jnp.zeros((_LANES,), jnp.float32))
        accw[...] = acc
        pltpu.sync_copy(accw, sh_acc.at[sid])
        plsc.subcore_barrier()

        @pl.when(sid == 0)
        def _():
            pltpu.sync_copy(sh_acc, accl)
            tot = jnp.zeros((_LANES,), jnp.float32)
            for r in range(_NT):
                tot = tot + plsc.load_gather(accl, [lane * 0 + r, lane])
            mean = jnp.sum(tot) * (1.0 / _PAIRS)
            ok = jnp.logical_and(n_ess > 0, n_non > 0)
            res = jnp.where(ok, mean, 0.0)
            outv[...] = jnp.full((_LANES,), 1.0, jnp.float32) * res
            pltpu.sync_copy(outv, out_hbm)


def kernel(probs, targets):
    bits_all = jnp.asarray(_RAW_BITS)
    tgt = targets.astype(jnp.int32)
    mesh = plsc.VectorSubcoreMesh(core_axis_name="c", subcore_axis_name="s", num_cores=1)
    f = pl.kernel(
        _body,
        out_type=jax.ShapeDtypeStruct((_LANES,), jnp.float32),
        mesh=mesh,
        compiler_params=pltpu.CompilerParams(needs_layout_passes=False),
        scratch_types=[
            pltpu.VMEM((_CHUNK,), jnp.float32),
            pltpu.VMEM((_CHUNK,), jnp.int32),
            pltpu.VMEM((2 * _CHUNK,), jnp.float32),
            pltpu.VMEM((_LANES,), jnp.int32),
            pltpu.VMEM((4 * _SPT,), jnp.int32),
            pltpu.VMEM((_NT, _LANES), jnp.int32),
            pltpu.VMEM((_LANES,), jnp.int32),
            pltpu.VMEM((_LANES,), jnp.int32),
            pltpu.VMEM((_LANES,), jnp.int32),
            pltpu.VMEM((_LANES,), jnp.int32),
            pltpu.VMEM((2 * _SPT,), jnp.int32),
            pltpu.VMEM((2 * _N,), jnp.float32),
            pltpu.VMEM((_NT * 2 * _SPT,), jnp.int32),
            pltpu.VMEM((_LANES,), jnp.float32),
            pltpu.VMEM((_NT, _LANES), jnp.float32),
            pltpu.VMEM((_LANES,), jnp.float32),
            pltpu.SemaphoreType.DMA,
            pltpu.SemaphoreType.DMA,
            pltpu.SemaphoreType.DMA,
            pltpu.VMEM_SHARED((2 * _N,), jnp.float32),
            pltpu.VMEM_SHARED((_NT, _LANES), jnp.int32),
            pltpu.VMEM_SHARED((_NT, _LANES), jnp.float32),
            pltpu.VMEM_SHARED((_NT * 2 * _SPT,), jnp.int32),
        ],
    )
    out = f(probs, tgt, bits_all)
    return out[0]


# speedup vs baseline: 1.2049x; 1.0082x over previous
import jax
import jax.numpy as jnp
import numpy as np
from jax import lax
from jax.experimental import pallas as pl
from jax.experimental.pallas import tpu as pltpu
from jax.experimental.pallas import tpu_sc as plsc

_MARGIN = 0.1
_N = 16384
_PAIRS = 8192
_LANES = 16
_NT = 16
_CHUNK = _N // _NT
_SPT = _PAIRS // _NT


def _rotl32(x, r):
    return ((x << np.uint32(r)) | (x >> np.uint32(32 - r))).astype(np.uint32)


def _threefry2x32(k1, k2, x0, x1):
    x0 = x0.astype(np.uint32).copy()
    x1 = x1.astype(np.uint32).copy()
    ks = [np.uint32(k1), np.uint32(k2),
          np.uint32(np.uint32(k1) ^ np.uint32(k2) ^ np.uint32(0x1BD11BDA))]
    rotations = [[13, 15, 26, 6], [17, 29, 16, 24]]
    x0 += ks[0]
    x1 += ks[1]
    for i in range(5):
        for r in rotations[i % 2]:
            x0 += x1
            x1 = _rotl32(x1, r)
            x1 ^= x0
        x0 += ks[(i + 1) % 3]
        x1 += np.uint32(ks[(i + 2) % 3] + np.uint32(i + 1))
    return x0, x1


def _fry_bits(k, n):
    i = np.arange(n, dtype=np.uint64)
    o0, o1 = _threefry2x32(k[0], k[1], (i >> np.uint64(32)).astype(np.uint32),
                           (i & np.uint64(0xFFFFFFFF)).astype(np.uint32))
    return o0 ^ o1


def _fry_split(k):
    o0, o1 = _threefry2x32(k[0], k[1], np.zeros(2, np.uint32),
                           np.arange(2, dtype=np.uint32))
    return (o0[0], o1[0]), (o0[1], o1[1])


def _sample_bits():
    sk1, sk2 = _fry_split((np.uint32(0), np.uint32(42)))
    out = []
    for k in (sk1, sk2):
        ra, rb = _fry_split(k)
        for kk in (ra, rb):
            out.append(_fry_bits(kk, _PAIRS).view(np.int32))
    return tuple(out)


def _packed_bits():
    ue, ve, un, vn = _sample_bits()
    tiles = []
    for w in range(_NT):
        sl = slice(w * _SPT, (w + 1) * _SPT)
        tiles.append(np.concatenate([ue[sl], ve[sl], un[sl], vn[sl]]))
    return np.concatenate(tiles)


_RAW_BITS = _packed_bits()


def _vmod(x, span_v, rinv_v):
    q1 = (x.astype(jnp.float32) * rinv_v).astype(jnp.int32)
    r = x - q1 * span_v
    q2 = (r.astype(jnp.float32) * rinv_v).astype(jnp.int32)
    r = r - q2 * span_v
    r = jnp.where(r >= span_v, r - span_v, r)
    r = jnp.where(r < 0, r + span_v, r)
    r = jnp.where(r < 0, r + span_v, r)
    return r


def _ridx(u, v, bf, bh, bg, span_v, rinv_v):
    uhi = lax.shift_right_logical(u, 16)
    ulo = lax.bitwise_and(u, 0xFFFF)
    vhi = lax.shift_right_logical(v, 16)
    vlo = lax.bitwise_and(v, 0xFFFF)
    p1 = _vmod(uhi * bg + ulo * bh, span_v, rinv_v)
    r = _vmod(p1 + vhi * bf + vlo, span_v, rinv_v)
    return jnp.minimum(jnp.maximum(r, 0), span_v - 1)


def _chunk_of(a, ce_tab):
    w = a * 0
    for s in (8, 4, 2, 1):
        probe = plsc.load_gather(ce_tab, [w + (s - 1)])
        w = w + jnp.where(probe <= a, s, 0)
    return w


def _body(probs_hbm, tgt_hbm, bits_hbm, out_hbm,
          pv, tv, locb, cntw, bits, cnts, ce_tab, cx_tab, cen_tab, cxn_tab,
          pos_loc, buf, accw, accl, outv, sem_bits, sem_buf, sem_in,
          sh_data, sh_cnt, sh_acc):
    cid = lax.axis_index("c")
    sid = lax.axis_index("s")

    @pl.when(cid == 0)
    def _():
        lane = lax.iota(jnp.int32, _LANES)

        cp_bits = pltpu.async_copy(bits_hbm.at[pl.ds(sid * 4 * _SPT, 4 * _SPT)],
                                   bits, sem_bits)

        base = sid * _CHUNK
        cp_p = pltpu.async_copy(probs_hbm.at[pl.ds(base, _CHUNK)], pv, sem_in)
        cp_t = pltpu.async_copy(tgt_hbm.at[pl.ds(base, _CHUNK)], tv, sem_in)
        cp_p.wait()
        cp_t.wait()

        def compact4(i, off_v):
            for k in range(8):
                j = i * 8 + k
                sl = pl.ds(j * _LANES, _LANES)
                t = tv[sl]
                p = pv[sl]
                m32 = jnp.where(t == 1, 1, 0)
                rank_e = plsc.cumsum(m32) - m32
                pc = plsc.all_reduce_population_count(t == 1)
                dest = jnp.where(m32 == 1, off_v + rank_e,
                                 (_CHUNK + j * _LANES) + lane - off_v - rank_e)
                plsc.store_scatter(locb, [dest], p)
                off_v = off_v + pc
            return off_v

        off_v = lax.fori_loop(0, _CHUNK // _LANES // 8, compact4,
                              jnp.zeros((_LANES,), jnp.int32))
        pltpu.sync_copy(locb, sh_data.at[pl.ds(sid * 2 * _CHUNK, 2 * _CHUNK)])
        cntw[...] = off_v
        pltpu.sync_copy(cntw, sh_cnt.at[sid])
        plsc.subcore_barrier()

        pltpu.async_copy(sh_data, buf, sem_buf)

        pltpu.sync_copy(sh_cnt, cnts)
        ce_vec = plsc.load_gather(cnts, [lane, lane * 0])
        ce_inc = plsc.cumsum(ce_vec)
        cx_exc = ce_inc - ce_vec
        cn_vec = _CHUNK - ce_vec
        cn_inc = plsc.cumsum(cn_vec)
        cxn_exc = cn_inc - cn_vec
        ce_tab[...] = ce_inc
        cx_tab[...] = cx_exc
        cen_tab[...] = cn_inc
        cxn_tab[...] = cxn_exc
        n_ess = ce_inc[_LANES - 1]
        n_non = _N - n_ess

        cp_bits.wait()

        se_v = lane * 0 + jnp.maximum(n_ess, 1)
        sn_v = lane * 0 + jnp.maximum(n_non, 1)
        rinv_e = 1.0 / se_v.astype(jnp.float32)
        rinv_n = 1.0 / sn_v.astype(jnp.float32)
        c64k = jnp.full((_LANES,), 65536, jnp.int32)
        bf_e = _vmod(c64k, se_v, rinv_e)
        bh_e = _vmod(bf_e * bf_e, se_v, rinv_e)
        bg_e = _vmod(lax.shift_left(bh_e, 16), se_v, rinv_e)
        bf_n = _vmod(c64k, sn_v, rinv_n)
        bh_n = _vmod(bf_n * bf_n, sn_v, rinv_n)
        bg_n = _vmod(lax.shift_left(bh_n, 16), sn_v, rinv_n)

        def samp4(i, carry):
            for k in range(4):
                v = i * 4 + k
                sl0 = pl.ds(v * _LANES, _LANES)
                a = _ridx(bits[sl0], bits[pl.ds(_SPT + v * _LANES, _LANES)],
                          bf_e, bh_e, bg_e, se_v, rinv_e)
                b = _ridx(bits[pl.ds(2 * _SPT + v * _LANES, _LANES)],
                          bits[pl.ds(3 * _SPT + v * _LANES, _LANES)],
                          bf_n, bh_n, bg_n, sn_v, rinv_n)
                wa = _chunk_of(a, ce_tab)
                pos_a = lax.shift_left(wa, 11) + a - plsc.load_gather(cx_tab, [wa])
                wb = _chunk_of(b, cen_tab)
                pos_b = (lax.shift_left(wb, 11) + _CHUNK + b
                         - plsc.load_gather(cxn_tab, [wb]))
                pos_loc[sl0] = pos_a
                pos_loc[pl.ds(_SPT + v * _LANES, _LANES)] = pos_b
            return carry

        lax.fori_loop(0, _SPT // _LANES // 4, samp4, jnp.int32(0))

        pltpu.make_async_copy(sh_data, buf, sem_buf).wait()

        def hinge8(i, a_):
            for k in range(8):
                o = (i * 8 + k) * _LANES
                ra = pos_loc[pl.ds(o, _LANES)]
                rb = pos_loc[pl.ds(_SPT + o, _LANES)]
                pe = plsc.load_gather(buf, [ra])
                pn = plsc.load_gather(buf, [rb])
                a_ = a_ + jnp.maximum(pn - pe + _MARGIN, 0.0)
            return a_

        acc = lax.fori_loop(0, _SPT // _LANES // 8, hinge8,
                            jnp.zeros((_LANES,), jnp.float32))
        accw[...] = acc
        pltpu.sync_copy(accw, sh_acc.at[sid])
        plsc.subcore_barrier()

        @pl.when(sid == 0)
        def _():
            pltpu.sync_copy(sh_acc, accl)
            tot = jnp.zeros((_LANES,), jnp.float32)
            for r in range(_NT):
                tot = tot + plsc.load_gather(accl, [lane * 0 + r, lane])
            mean = jnp.sum(tot) * (1.0 / _PAIRS)
            ok = jnp.logical_and(n_ess > 0, n_non > 0)
            res = jnp.where(ok, mean, 0.0)
            outv[...] = jnp.full((_LANES,), 1.0, jnp.float32) * res
            pltpu.sync_copy(outv, out_hbm)


def kernel(probs, targets):
    bits_all = jnp.asarray(_RAW_BITS)
    tgt = targets.astype(jnp.int32)
    mesh = plsc.VectorSubcoreMesh(core_axis_name="c", subcore_axis_name="s", num_cores=1)
    f = pl.kernel(
        _body,
        out_type=jax.ShapeDtypeStruct((_LANES,), jnp.float32),
        mesh=mesh,
        compiler_params=pltpu.CompilerParams(needs_layout_passes=False),
        scratch_types=[
            pltpu.VMEM((_CHUNK,), jnp.float32),
            pltpu.VMEM((_CHUNK,), jnp.int32),
            pltpu.VMEM((2 * _CHUNK,), jnp.float32),
            pltpu.VMEM((_LANES,), jnp.int32),
            pltpu.VMEM((4 * _SPT,), jnp.int32),
            pltpu.VMEM((_NT, _LANES), jnp.int32),
            pltpu.VMEM((_LANES,), jnp.int32),
            pltpu.VMEM((_LANES,), jnp.int32),
            pltpu.VMEM((_LANES,), jnp.int32),
            pltpu.VMEM((_LANES,), jnp.int32),
            pltpu.VMEM((2 * _SPT,), jnp.int32),
            pltpu.VMEM((2 * _N,), jnp.float32),
            pltpu.VMEM((_LANES,), jnp.float32),
            pltpu.VMEM((_NT, _LANES), jnp.float32),
            pltpu.VMEM((_LANES,), jnp.float32),
            pltpu.SemaphoreType.DMA,
            pltpu.SemaphoreType.DMA,
            pltpu.SemaphoreType.DMA,
            pltpu.VMEM_SHARED((2 * _N,), jnp.float32),
            pltpu.VMEM_SHARED((_NT, _LANES), jnp.int32),
            pltpu.VMEM_SHARED((_NT, _LANES), jnp.float32),
        ],
    )
    out = f(probs, tgt, bits_all)
    return out[0]
